# baseline (speedup 1.0000x reference)
import functools

import jax
import jax.numpy as jnp
from jax.experimental import pallas as pl
from jax.experimental.pallas import tpu as pltpu

_BIG = 1e30
_HI = jax.lax.Precision.HIGHEST


def _dist(qc, qn, pcT, pn):
    cross = jnp.dot(qc, pcT, preferred_element_type=jnp.float32)
    return (qn + cross) + pn


def _down_body(nk, qc_ref, qn_ref, pcT_ref, pn_ref, pf2d_ref, hivec_ref,
               seg_ref, qsub_ref, w1_ref, b1_ref, w2_ref, b2_ref,
               out_ref, d_ref):
    d_ref[...] = _dist(qc_ref[0], qn_ref[0], pcT_ref[0], pn_ref[0])
    bs, n = d_ref.shape
    iota = jax.lax.broadcasted_iota(jnp.int32, d_ref.shape, 1).astype(jnp.float32)
    iota128 = jax.lax.broadcasted_iota(jnp.int32, (bs, 128), 1)
    qsub = qsub_ref[0]
    pf2d = pf2d_ref[0]
    hivec = hivec_ref[...]
    seg = seg_ref[...]
    w1 = w1_ref[...]
    b1 = b1_ref[...]
    w2 = w2_ref[...]
    b2 = b2_ref[...]

    nc128 = n // 128
    lane_f = jax.lax.broadcasted_iota(jnp.int32, (bs, 128), 1).astype(jnp.float32)

    def body(_, acc):
        dd = d_ref[...]
        acc_v = dd[:, 0:128]
        acc_c = jnp.zeros((bs, 128), jnp.float32)
        for c in range(1, nc128):
            s = dd[:, c * 128:(c + 1) * 128]
            upd = s < acc_v
            acc_v = jnp.where(upd, s, acc_v)
            acc_c = jnp.where(upd, jnp.float32(c), acc_c)
        mv = jnp.min(acc_v, axis=1, keepdims=True)
        comp = acc_c * 128.0 + lane_f
        tl = jnp.where(acc_v <= mv, comp, jnp.float32(n))
        j = jnp.min(tl, axis=1, keepdims=True)
        d_ref[...] = jnp.where(iota == j, _BIG, dd)
        ji = j.astype(jnp.int32)
        jhi = jax.lax.shift_right_logical(ji, 7)
        jlo = jax.lax.bitwise_and(ji, 127)
        oh_lo = (iota128 == jlo).astype(jnp.float32)
        t = jnp.dot(oh_lo, pf2d, preferred_element_type=jnp.float32,
                    precision=_HI)
        t = jnp.where(hivec == jhi, t, 0.0)
        g = jnp.dot(t, seg, preferred_element_type=jnp.float32,
                    precision=_HI) - qsub
        h = jnp.maximum(jnp.dot(g, w1, preferred_element_type=jnp.float32) + b1, 0.0)
        h = jnp.maximum(jnp.dot(h, w2, preferred_element_type=jnp.float32) + b2, 0.0)
        return jnp.maximum(acc, h)

    acc0 = jnp.zeros(out_ref.shape[1:], jnp.float32)
    out_ref[0] = jax.lax.fori_loop(0, nk, body, acc0)


def _down(qc, qn, pcT, pn, pf, qsub, w1T, b1, w2T, b2, nk, bs):
    B, S, _ = qc.shape
    N, CF = pf.shape[1], pf.shape[2]
    C1, C2 = w1T.shape[1], w2T.shape[1]
    nc = N // 128
    pf2d = pf.reshape(B, nc, 128, CF).transpose(0, 2, 1, 3).reshape(B, 128, nc * CF)
    hivec = (jnp.arange(nc * CF, dtype=jnp.int32) // CF)[None]
    seg = (jnp.arange(nc * CF)[:, None] % CF ==
           jnp.arange(CF)[None, :]).astype(jnp.float32)
    return pl.pallas_call(
        functools.partial(_down_body, nk),
        grid=(B, S // bs),
        in_specs=[
            pl.BlockSpec((1, bs, 8), lambda b, s: (b, s, 0)),
            pl.BlockSpec((1, bs, 1), lambda b, s: (b, s, 0)),
            pl.BlockSpec((1, 8, N), lambda b, s: (b, 0, 0)),
            pl.BlockSpec((1, 1, N), lambda b, s: (b, 0, 0)),
            pl.BlockSpec((1, 128, nc * CF), lambda b, s: (b, 0, 0)),
            pl.BlockSpec((1, nc * CF), lambda b, s: (0, 0)),
            pl.BlockSpec((nc * CF, CF), lambda b, s: (0, 0)),
            pl.BlockSpec((1, bs, CF), lambda b, s: (b, s, 0)),
            pl.BlockSpec((CF, C1), lambda b, s: (0, 0)),
            pl.BlockSpec((1, C1), lambda b, s: (0, 0)),
            pl.BlockSpec((C1, C2), lambda b, s: (0, 0)),
            pl.BlockSpec((1, C2), lambda b, s: (0, 0)),
        ],
        out_specs=pl.BlockSpec((1, bs, C2), lambda b, s: (b, s, 0)),
        out_shape=jax.ShapeDtypeStruct((B, S, C2), jnp.float32),
        scratch_shapes=[pltpu.VMEM((bs, N), jnp.float32)],
    )(qc, qn, pcT, pn, pf2d, hivec, seg, qsub, w1T, b1, w2T, b2)


def _fp_interp(qc_ref, qn_ref, pcT_ref, pn_ref, fs_ref):
    d = _dist(qc_ref[0], qn_ref[0], pcT_ref[0], pn_ref[0])
    n = d.shape[1]
    iota = jax.lax.broadcasted_iota(jnp.int32, d.shape, 1).astype(jnp.float32)
    ws = jnp.zeros_like(d)
    wtot = jnp.zeros((d.shape[0], 1), jnp.float32)
    for _ in range(3):
        m = jnp.min(d, axis=1, keepdims=True)
        ti = jnp.where(d <= m, iota, jnp.float32(n))
        j = jnp.min(ti, axis=1, keepdims=True)
        oh = (ti == j).astype(jnp.float32)
        w = 1.0 / jnp.maximum(m, 1e-10)
        ws = ws + w * oh
        wtot = wtot + w
        d = d + oh * _BIG
    interp = jnp.dot(ws, fs_ref[0], preferred_element_type=jnp.float32,
                     precision=_HI)
    return interp / wtot


def _fp_body(qc_ref, qn_ref, pcT_ref, pn_ref, fs_ref, skip_ref,
             uwi_ref, uws_ref, ub_ref, out_ref):
    interp = _fp_interp(qc_ref, qn_ref, pcT_ref, pn_ref, fs_ref)
    f = (jnp.dot(interp, uwi_ref[...], preferred_element_type=jnp.float32)
         + jnp.dot(skip_ref[0], uws_ref[...], preferred_element_type=jnp.float32)
         + ub_ref[...])
    out_ref[0] = jnp.maximum(f, 0.0)


def _fp_head_body(qc_ref, qn_ref, pcT_ref, pn_ref, fs_ref, skip_ref,
                  uwi_ref, uws_ref, ub_ref,
                  fw1_ref, fb1_ref, fw2_ref, fb2_ref, out_ref):
    interp = _fp_interp(qc_ref, qn_ref, pcT_ref, pn_ref, fs_ref)
    f = (jnp.dot(interp, uwi_ref[...], preferred_element_type=jnp.float32)
         + jnp.dot(skip_ref[0], uws_ref[...], preferred_element_type=jnp.float32)
         + ub_ref[...])
    f = jnp.maximum(f, 0.0)
    h = jnp.maximum(jnp.dot(f, fw1_ref[...], preferred_element_type=jnp.float32)
                    + fb1_ref[...], 0.0)
    out_ref[0] = (jnp.dot(h, fw2_ref[...], preferred_element_type=jnp.float32)
                  + fb2_ref[...])


def _fp(qc, qn, pcT, pn, fs, skip, uwiT, uwsT, ub, bs, head=None):
    B, S, _ = qc.shape
    Ns, C = fs.shape[1], fs.shape[2]
    CS = skip.shape[2]
    CO = uwiT.shape[1]
    extra_in, extra_specs = (), []
    body = _fp_body
    if head is not None:
        fw1T, fb1, fw2T, fb2 = head
        CO = fw2T.shape[1]
        extra_in = (fw1T, fb1, fw2T, fb2)
        extra_specs = [
            pl.BlockSpec(fw1T.shape, lambda b, s: (0, 0)),
            pl.BlockSpec(fb1.shape, lambda b, s: (0, 0)),
            pl.BlockSpec(fw2T.shape, lambda b, s: (0, 0)),
            pl.BlockSpec(fb2.shape, lambda b, s: (0, 0)),
        ]
        body = _fp_head_body
    return pl.pallas_call(
        body,
        grid=(B, S // bs),
        in_specs=[
            pl.BlockSpec((1, bs, 8), lambda b, s: (b, s, 0)),
            pl.BlockSpec((1, bs, 1), lambda b, s: (b, s, 0)),
            pl.BlockSpec((1, 8, Ns), lambda b, s: (b, 0, 0)),
            pl.BlockSpec((1, 1, Ns), lambda b, s: (b, 0, 0)),
            pl.BlockSpec((1, Ns, C), lambda b, s: (b, 0, 0)),
            pl.BlockSpec((1, bs, CS), lambda b, s: (b, s, 0)),
            pl.BlockSpec(uwiT.shape, lambda b, s: (0, 0)),
            pl.BlockSpec(uwsT.shape, lambda b, s: (0, 0)),
            pl.BlockSpec(ub.shape, lambda b, s: (0, 0)),
        ] + extra_specs,
        out_specs=pl.BlockSpec((1, bs, CO), lambda b, s: (b, s, 0)),
        out_shape=jax.ShapeDtypeStruct((B, S, CO), jnp.float32),
    )(qc, qn, pcT, pn, fs, skip, uwiT, uwsT, ub, *extra_in)


def _aug(p):
    nrm = jnp.sum(p ** 2, axis=2, keepdims=True)
    z5 = jnp.zeros(p.shape[:2] + (5,), p.dtype)
    qc = jnp.concatenate([-2.0 * p, z5], axis=2)
    pcT = jnp.concatenate([p, z5], axis=2).transpose(0, 2, 1)
    return qc, nrm, pcT, nrm.transpose(0, 2, 1)


def kernel(xyz, feat, d1w1, d1b1, d1w2, d1b2, d2w1, d2b1, d2w2, d2b2,
           u1w, u1b, u2w, u2b, fw1, fb1, fw2, fb2):
    B, _, _, N = xyz.shape
    pts = xyz[:, 0].transpose(0, 2, 1)
    f0 = feat[:, 0].transpose(0, 2, 1)

    qc0, qn0, pcT0, pn0 = _aug(pts)
    pf1 = jnp.concatenate([f0, pts], axis=2)
    qsub1 = jnp.concatenate(
        [jnp.zeros((B, N, 1), jnp.float32), pts], axis=2)[:, ::4]

    nf1 = _down(qc0[:, ::4], qn0[:, ::4], pcT0, pn0, pf1, qsub1,
                d1w1.T, d1b1[None], d1w2.T, d1b2[None], nk=32, bs=256)

    xyz1 = pts[:, ::4]
    qc1, qn1, pcT1, pn1 = _aug(xyz1)
    pf2 = jnp.concatenate([nf1, xyz1], axis=2)
    qsub2 = jnp.concatenate(
        [jnp.zeros((B, 2048, 64), jnp.float32), xyz1], axis=2)[:, ::4]

    nf2 = _down(qc1[:, ::4], qn1[:, ::4], pcT1, pn1, pf2, qsub2,
                d2w1.T, d2b1[None], d2w2.T, d2b2[None], nk=32, bs=512)

    _, _, pcT2, pn2 = _aug(xyz1[:, ::4])
    f1 = _fp(qc1, qn1, pcT2, pn2, nf2, nf1,
             u1w[:, :128].T, u1w[:, 128:].T, u1b[None], bs=512)

    skip0 = jnp.concatenate([f0, jnp.zeros((B, N, 7), jnp.float32)], axis=2)
    uws2 = jnp.pad(u2w[:, 128:], ((0, 0), (0, 7)))
    fw2p = jnp.pad(fw2, ((0, 5), (0, 0)))
    fb2p = jnp.pad(fb2, (0, 5))
    out = _fp(qc0, qn0, pcT1, pn1, f1, skip0,
              u2w[:, :128].T, uws2.T, u2b[None], bs=512,
              head=(fw1.T, fb1[None], fw2p.T, fb2p[None]))

    return out[:, :, :3].transpose(0, 2, 1)


# speedup vs baseline: 1.1030x; 1.1030x over previous
import functools

import jax
import jax.numpy as jnp
from jax.experimental import pallas as pl
from jax.experimental.pallas import tpu as pltpu

_BIG = 1e30
_HI = jax.lax.Precision.HIGHEST


def _dist(qc, qn, pcT, pn):
    cross = jnp.dot(qc, pcT, preferred_element_type=jnp.float32)
    return (qn + cross) + pn


def _down_body(nk, qc_ref, qn_ref, pcT_ref, pn_ref, pf2d_ref, hivec_ref,
               seg_ref, qsub_ref, w1_ref, b1_ref, w2_ref, b2_ref,
               out_ref, d_ref):
    d_ref[...] = _dist(qc_ref[0], qn_ref[0], pcT_ref[0], pn_ref[0])
    bs, n = d_ref.shape
    iota = jax.lax.broadcasted_iota(jnp.int32, d_ref.shape, 1).astype(jnp.float32)
    iota128 = jax.lax.broadcasted_iota(jnp.int32, (bs, 128), 1)
    qsub = qsub_ref[0]
    pf2d = pf2d_ref[0]
    hivec = hivec_ref[...]
    seg = seg_ref[...]
    w1 = w1_ref[...]
    b1 = b1_ref[...]
    w2 = w2_ref[...]
    b2 = b2_ref[...]

    def body(_, carry):
        acc, m = carry
        dd = d_ref[...]
        ti = jnp.where(dd <= m, iota, jnp.float32(n))
        j = jnp.min(ti, axis=1, keepdims=True)
        masked = jnp.where(ti == j, _BIG, dd)
        d_ref[...] = masked
        m = jnp.min(masked, axis=1, keepdims=True)
        ji = j.astype(jnp.int32)
        jhi = jax.lax.shift_right_logical(ji, 7)
        jlo = jax.lax.bitwise_and(ji, 127)
        oh_lo = (iota128 == jlo).astype(jnp.float32)
        t = jnp.dot(oh_lo, pf2d, preferred_element_type=jnp.float32,
                    precision=_HI)
        t = jnp.where(hivec == jhi, t, 0.0)
        g = jnp.dot(t, seg, preferred_element_type=jnp.float32,
                    precision=_HI) - qsub
        h = jnp.maximum(jnp.dot(g, w1, preferred_element_type=jnp.float32) + b1, 0.0)
        h = jnp.maximum(jnp.dot(h, w2, preferred_element_type=jnp.float32) + b2, 0.0)
        return jnp.maximum(acc, h), m

    acc0 = jnp.zeros(out_ref.shape[1:], jnp.float32)
    m0 = jnp.min(d_ref[...], axis=1, keepdims=True)
    out_ref[0] = jax.lax.fori_loop(0, nk, body, (acc0, m0))[0]


def _down(qc, qn, pcT, pn, pf, qsub, w1T, b1, w2T, b2, nk, bs):
    B, S, _ = qc.shape
    N, CF = pf.shape[1], pf.shape[2]
    C1, C2 = w1T.shape[1], w2T.shape[1]
    nc = N // 128
    pf2d = pf.reshape(B, nc, 128, CF).transpose(0, 2, 1, 3).reshape(B, 128, nc * CF)
    hivec = (jnp.arange(nc * CF, dtype=jnp.int32) // CF)[None]
    seg = (jnp.arange(nc * CF)[:, None] % CF ==
           jnp.arange(CF)[None, :]).astype(jnp.float32)
    return pl.pallas_call(
        functools.partial(_down_body, nk),
        grid=(B, S // bs),
        in_specs=[
            pl.BlockSpec((1, bs, 8), lambda b, s: (b, s, 0)),
            pl.BlockSpec((1, bs, 1), lambda b, s: (b, s, 0)),
            pl.BlockSpec((1, 8, N), lambda b, s: (b, 0, 0)),
            pl.BlockSpec((1, 1, N), lambda b, s: (b, 0, 0)),
            pl.BlockSpec((1, 128, nc * CF), lambda b, s: (b, 0, 0)),
            pl.BlockSpec((1, nc * CF), lambda b, s: (0, 0)),
            pl.BlockSpec((nc * CF, CF), lambda b, s: (0, 0)),
            pl.BlockSpec((1, bs, CF), lambda b, s: (b, s, 0)),
            pl.BlockSpec((CF, C1), lambda b, s: (0, 0)),
            pl.BlockSpec((1, C1), lambda b, s: (0, 0)),
            pl.BlockSpec((C1, C2), lambda b, s: (0, 0)),
            pl.BlockSpec((1, C2), lambda b, s: (0, 0)),
        ],
        out_specs=pl.BlockSpec((1, bs, C2), lambda b, s: (b, s, 0)),
        out_shape=jax.ShapeDtypeStruct((B, S, C2), jnp.float32),
        scratch_shapes=[pltpu.VMEM((bs, N), jnp.float32)],
    )(qc, qn, pcT, pn, pf2d, hivec, seg, qsub, w1T, b1, w2T, b2)


def _fp_interp(qc_ref, qn_ref, pcT_ref, pn_ref, fs_ref):
    d = _dist(qc_ref[0], qn_ref[0], pcT_ref[0], pn_ref[0])
    n = d.shape[1]
    iota = jax.lax.broadcasted_iota(jnp.int32, d.shape, 1).astype(jnp.float32)
    ws = jnp.zeros_like(d)
    wtot = jnp.zeros((d.shape[0], 1), jnp.float32)
    for _ in range(3):
        m = jnp.min(d, axis=1, keepdims=True)
        ti = jnp.where(d <= m, iota, jnp.float32(n))
        j = jnp.min(ti, axis=1, keepdims=True)
        oh = (ti == j).astype(jnp.float32)
        w = 1.0 / jnp.maximum(m, 1e-10)
        ws = ws + w * oh
        wtot = wtot + w
        d = d + oh * _BIG
    interp = jnp.dot(ws, fs_ref[0], preferred_element_type=jnp.float32,
                     precision=_HI)
    return interp / wtot


def _fp_body(qc_ref, qn_ref, pcT_ref, pn_ref, fs_ref, skip_ref,
             uwi_ref, uws_ref, ub_ref, out_ref):
    interp = _fp_interp(qc_ref, qn_ref, pcT_ref, pn_ref, fs_ref)
    f = (jnp.dot(interp, uwi_ref[...], preferred_element_type=jnp.float32)
         + jnp.dot(skip_ref[0], uws_ref[...], preferred_element_type=jnp.float32)
         + ub_ref[...])
    out_ref[0] = jnp.maximum(f, 0.0)


def _fp_head_body(qc_ref, qn_ref, pcT_ref, pn_ref, fs_ref, skip_ref,
                  uwi_ref, uws_ref, ub_ref,
                  fw1_ref, fb1_ref, fw2_ref, fb2_ref, out_ref):
    interp = _fp_interp(qc_ref, qn_ref, pcT_ref, pn_ref, fs_ref)
    f = (jnp.dot(interp, uwi_ref[...], preferred_element_type=jnp.float32)
         + jnp.dot(skip_ref[0], uws_ref[...], preferred_element_type=jnp.float32)
         + ub_ref[...])
    f = jnp.maximum(f, 0.0)
    h = jnp.maximum(jnp.dot(f, fw1_ref[...], preferred_element_type=jnp.float32)
                    + fb1_ref[...], 0.0)
    out_ref[0] = (jnp.dot(h, fw2_ref[...], preferred_element_type=jnp.float32)
                  + fb2_ref[...])


def _fp(qc, qn, pcT, pn, fs, skip, uwiT, uwsT, ub, bs, head=None):
    B, S, _ = qc.shape
    Ns, C = fs.shape[1], fs.shape[2]
    CS = skip.shape[2]
    CO = uwiT.shape[1]
    extra_in, extra_specs = (), []
    body = _fp_body
    if head is not None:
        fw1T, fb1, fw2T, fb2 = head
        CO = fw2T.shape[1]
        extra_in = (fw1T, fb1, fw2T, fb2)
        extra_specs = [
            pl.BlockSpec(fw1T.shape, lambda b, s: (0, 0)),
            pl.BlockSpec(fb1.shape, lambda b, s: (0, 0)),
            pl.BlockSpec(fw2T.shape, lambda b, s: (0, 0)),
            pl.BlockSpec(fb2.shape, lambda b, s: (0, 0)),
        ]
        body = _fp_head_body
    return pl.pallas_call(
        body,
        grid=(B, S // bs),
        in_specs=[
            pl.BlockSpec((1, bs, 8), lambda b, s: (b, s, 0)),
            pl.BlockSpec((1, bs, 1), lambda b, s: (b, s, 0)),
            pl.BlockSpec((1, 8, Ns), lambda b, s: (b, 0, 0)),
            pl.BlockSpec((1, 1, Ns), lambda b, s: (b, 0, 0)),
            pl.BlockSpec((1, Ns, C), lambda b, s: (b, 0, 0)),
            pl.BlockSpec((1, bs, CS), lambda b, s: (b, s, 0)),
            pl.BlockSpec(uwiT.shape, lambda b, s: (0, 0)),
            pl.BlockSpec(uwsT.shape, lambda b, s: (0, 0)),
            pl.BlockSpec(ub.shape, lambda b, s: (0, 0)),
        ] + extra_specs,
        out_specs=pl.BlockSpec((1, bs, CO), lambda b, s: (b, s, 0)),
        out_shape=jax.ShapeDtypeStruct((B, S, CO), jnp.float32),
    )(qc, qn, pcT, pn, fs, skip, uwiT, uwsT, ub, *extra_in)


def _aug(p):
    nrm = jnp.sum(p ** 2, axis=2, keepdims=True)
    z5 = jnp.zeros(p.shape[:2] + (5,), p.dtype)
    qc = jnp.concatenate([-2.0 * p, z5], axis=2)
    pcT = jnp.concatenate([p, z5], axis=2).transpose(0, 2, 1)
    return qc, nrm, pcT, nrm.transpose(0, 2, 1)


def kernel(xyz, feat, d1w1, d1b1, d1w2, d1b2, d2w1, d2b1, d2w2, d2b2,
           u1w, u1b, u2w, u2b, fw1, fb1, fw2, fb2):
    B, _, _, N = xyz.shape
    pts = xyz[:, 0].transpose(0, 2, 1)
    f0 = feat[:, 0].transpose(0, 2, 1)

    qc0, qn0, pcT0, pn0 = _aug(pts)
    pf1 = jnp.concatenate([f0, pts], axis=2)
    qsub1 = jnp.concatenate(
        [jnp.zeros((B, N, 1), jnp.float32), pts], axis=2)[:, ::4]

    nf1 = _down(qc0[:, ::4], qn0[:, ::4], pcT0, pn0, pf1, qsub1,
                d1w1.T, d1b1[None], d1w2.T, d1b2[None], nk=32, bs=256)

    xyz1 = pts[:, ::4]
    qc1, qn1, pcT1, pn1 = _aug(xyz1)
    pf2 = jnp.concatenate([nf1, xyz1], axis=2)
    qsub2 = jnp.concatenate(
        [jnp.zeros((B, 2048, 64), jnp.float32), xyz1], axis=2)[:, ::4]

    nf2 = _down(qc1[:, ::4], qn1[:, ::4], pcT1, pn1, pf2, qsub2,
                d2w1.T, d2b1[None], d2w2.T, d2b2[None], nk=32, bs=512)

    _, _, pcT2, pn2 = _aug(xyz1[:, ::4])
    f1 = _fp(qc1, qn1, pcT2, pn2, nf2, nf1,
             u1w[:, :128].T, u1w[:, 128:].T, u1b[None], bs=512)

    skip0 = jnp.concatenate([f0, jnp.zeros((B, N, 7), jnp.float32)], axis=2)
    uws2 = jnp.pad(u2w[:, 128:], ((0, 0), (0, 7)))
    fw2p = jnp.pad(fw2, ((0, 5), (0, 0)))
    fb2p = jnp.pad(fb2, (0, 5))
    out = _fp(qc0, qn0, pcT1, pn1, f1, skip0,
              u2w[:, :128].T, uws2.T, u2b[None], bs=512,
              head=(fw1.T, fb1[None], fw2p.T, fb2p[None]))

    return out[:, :, :3].transpose(0, 2, 1)


# speedup vs baseline: 1.3740x; 1.2457x over previous
import functools

import jax
import jax.numpy as jnp
from jax.experimental import pallas as pl
from jax.experimental.pallas import tpu as pltpu

_BIG = 1e30
_HI = jax.lax.Precision.HIGHEST


def _dist(qc, qn, pcT, pn):
    cross = jnp.dot(qc, pcT, preferred_element_type=jnp.float32)
    return (qn + cross) + pn


def _down_body(nk, qc_ref, qn_ref, pcT_ref, pn_ref, pfhi_ref, pflo_ref,
               hivec_ref, seg_ref, qsub_ref, w1_ref, b1_ref, w2_ref, b2_ref,
               out_ref, d_ref):
    d_ref[...] = _dist(qc_ref[0], qn_ref[0], pcT_ref[0], pn_ref[0])
    bs, n = d_ref.shape
    iota = jax.lax.broadcasted_iota(jnp.int32, d_ref.shape, 1).astype(jnp.float32)
    iota128 = jax.lax.broadcasted_iota(jnp.int32, (bs, 128), 1)
    qsub = qsub_ref[0]
    pfhi = pfhi_ref[0]
    pflo = pflo_ref[0]
    hivec = hivec_ref[...]
    seg = seg_ref[...]
    w1 = w1_ref[...]
    b1 = b1_ref[...]
    w2 = w2_ref[...]
    b2 = b2_ref[...]

    def body(_, carry):
        acc, m = carry
        dd = d_ref[...]
        ti = jnp.where(dd <= m, iota, jnp.float32(n))
        j = jnp.min(ti, axis=1, keepdims=True)
        masked = jnp.where(ti == j, _BIG, dd)
        d_ref[...] = masked
        m = jnp.min(masked, axis=1, keepdims=True)
        ji = j.astype(jnp.int32)
        jhi = jax.lax.shift_right_logical(ji, 7)
        jlo = jax.lax.bitwise_and(ji, 127)
        oh_lo = (iota128 == jlo).astype(jnp.float32)
        cm = hivec == jhi
        thi = jnp.where(cm, jnp.dot(oh_lo, pfhi,
                                    preferred_element_type=jnp.float32), 0.0)
        tlo = jnp.where(cm, jnp.dot(oh_lo, pflo,
                                    preferred_element_type=jnp.float32), 0.0)
        g = (jnp.dot(thi, seg, preferred_element_type=jnp.float32)
             + jnp.dot(tlo, seg, preferred_element_type=jnp.float32)) - qsub
        h = jnp.maximum(jnp.dot(g, w1, preferred_element_type=jnp.float32) + b1, 0.0)
        h = jnp.maximum(jnp.dot(h, w2, preferred_element_type=jnp.float32) + b2, 0.0)
        return jnp.maximum(acc, h), m

    acc0 = jnp.zeros(out_ref.shape[1:], jnp.float32)
    m0 = jnp.min(d_ref[...], axis=1, keepdims=True)
    out_ref[0] = jax.lax.fori_loop(0, nk, body, (acc0, m0))[0]


def _down(qc, qn, pcT, pn, pf, qsub, w1T, b1, w2T, b2, nk, bs):
    B, S, _ = qc.shape
    N, CF = pf.shape[1], pf.shape[2]
    C1, C2 = w1T.shape[1], w2T.shape[1]
    nc = N // 128
    pf2d = pf.reshape(B, nc, 128, CF).transpose(0, 2, 1, 3).reshape(B, 128, nc * CF)
    pfhi = pf2d.astype(jnp.bfloat16).astype(jnp.float32)
    pflo = pf2d - pfhi
    hivec = (jnp.arange(nc * CF, dtype=jnp.int32) // CF)[None]
    seg = (jnp.arange(nc * CF)[:, None] % CF ==
           jnp.arange(CF)[None, :]).astype(jnp.float32)
    return pl.pallas_call(
        functools.partial(_down_body, nk),
        grid=(B, S // bs),
        in_specs=[
            pl.BlockSpec((1, bs, 8), lambda b, s: (b, s, 0)),
            pl.BlockSpec((1, bs, 1), lambda b, s: (b, s, 0)),
            pl.BlockSpec((1, 8, N), lambda b, s: (b, 0, 0)),
            pl.BlockSpec((1, 1, N), lambda b, s: (b, 0, 0)),
            pl.BlockSpec((1, 128, nc * CF), lambda b, s: (b, 0, 0)),
            pl.BlockSpec((1, 128, nc * CF), lambda b, s: (b, 0, 0)),
            pl.BlockSpec((1, nc * CF), lambda b, s: (0, 0)),
            pl.BlockSpec((nc * CF, CF), lambda b, s: (0, 0)),
            pl.BlockSpec((1, bs, CF), lambda b, s: (b, s, 0)),
            pl.BlockSpec((CF, C1), lambda b, s: (0, 0)),
            pl.BlockSpec((1, C1), lambda b, s: (0, 0)),
            pl.BlockSpec((C1, C2), lambda b, s: (0, 0)),
            pl.BlockSpec((1, C2), lambda b, s: (0, 0)),
        ],
        out_specs=pl.BlockSpec((1, bs, C2), lambda b, s: (b, s, 0)),
        out_shape=jax.ShapeDtypeStruct((B, S, C2), jnp.float32),
        scratch_shapes=[pltpu.VMEM((bs, N), jnp.float32)],
    )(qc, qn, pcT, pn, pfhi, pflo, hivec, seg, qsub, w1T, b1, w2T, b2)


def _fp_interp(qc_ref, qn_ref, pcT_ref, pn_ref, fs_ref):
    d = _dist(qc_ref[0], qn_ref[0], pcT_ref[0], pn_ref[0])
    n = d.shape[1]
    iota = jax.lax.broadcasted_iota(jnp.int32, d.shape, 1).astype(jnp.float32)
    ws = jnp.zeros_like(d)
    wtot = jnp.zeros((d.shape[0], 1), jnp.float32)
    for _ in range(3):
        m = jnp.min(d, axis=1, keepdims=True)
        ti = jnp.where(d <= m, iota, jnp.float32(n))
        j = jnp.min(ti, axis=1, keepdims=True)
        oh = (ti == j).astype(jnp.float32)
        w = 1.0 / jnp.maximum(m, 1e-10)
        ws = ws + w * oh
        wtot = wtot + w
        d = d + oh * _BIG
    interp = jnp.dot(ws, fs_ref[0], preferred_element_type=jnp.float32,
                     precision=_HI)
    return interp / wtot


def _fp_body(qc_ref, qn_ref, pcT_ref, pn_ref, fs_ref, skip_ref,
             uwi_ref, uws_ref, ub_ref, out_ref):
    interp = _fp_interp(qc_ref, qn_ref, pcT_ref, pn_ref, fs_ref)
    f = (jnp.dot(interp, uwi_ref[...], preferred_element_type=jnp.float32)
         + jnp.dot(skip_ref[0], uws_ref[...], preferred_element_type=jnp.float32)
         + ub_ref[...])
    out_ref[0] = jnp.maximum(f, 0.0)


def _fp_head_body(qc_ref, qn_ref, pcT_ref, pn_ref, fs_ref, skip_ref,
                  uwi_ref, uws_ref, ub_ref,
                  fw1_ref, fb1_ref, fw2_ref, fb2_ref, out_ref):
    interp = _fp_interp(qc_ref, qn_ref, pcT_ref, pn_ref, fs_ref)
    f = (jnp.dot(interp, uwi_ref[...], preferred_element_type=jnp.float32)
         + jnp.dot(skip_ref[0], uws_ref[...], preferred_element_type=jnp.float32)
         + ub_ref[...])
    f = jnp.maximum(f, 0.0)
    h = jnp.maximum(jnp.dot(f, fw1_ref[...], preferred_element_type=jnp.float32)
                    + fb1_ref[...], 0.0)
    out_ref[0] = (jnp.dot(h, fw2_ref[...], preferred_element_type=jnp.float32)
                  + fb2_ref[...])


def _fp(qc, qn, pcT, pn, fs, skip, uwiT, uwsT, ub, bs, head=None):
    B, S, _ = qc.shape
    Ns, C = fs.shape[1], fs.shape[2]
    CS = skip.shape[2]
    CO = uwiT.shape[1]
    extra_in, extra_specs = (), []
    body = _fp_body
    if head is not None:
        fw1T, fb1, fw2T, fb2 = head
        CO = fw2T.shape[1]
        extra_in = (fw1T, fb1, fw2T, fb2)
        extra_specs = [
            pl.BlockSpec(fw1T.shape, lambda b, s: (0, 0)),
            pl.BlockSpec(fb1.shape, lambda b, s: (0, 0)),
            pl.BlockSpec(fw2T.shape, lambda b, s: (0, 0)),
            pl.BlockSpec(fb2.shape, lambda b, s: (0, 0)),
        ]
        body = _fp_head_body
    return pl.pallas_call(
        body,
        grid=(B, S // bs),
        in_specs=[
            pl.BlockSpec((1, bs, 8), lambda b, s: (b, s, 0)),
            pl.BlockSpec((1, bs, 1), lambda b, s: (b, s, 0)),
            pl.BlockSpec((1, 8, Ns), lambda b, s: (b, 0, 0)),
            pl.BlockSpec((1, 1, Ns), lambda b, s: (b, 0, 0)),
            pl.BlockSpec((1, Ns, C), lambda b, s: (b, 0, 0)),
            pl.BlockSpec((1, bs, CS), lambda b, s: (b, s, 0)),
            pl.BlockSpec(uwiT.shape, lambda b, s: (0, 0)),
            pl.BlockSpec(uwsT.shape, lambda b, s: (0, 0)),
            pl.BlockSpec(ub.shape, lambda b, s: (0, 0)),
        ] + extra_specs,
        out_specs=pl.BlockSpec((1, bs, CO), lambda b, s: (b, s, 0)),
        out_shape=jax.ShapeDtypeStruct((B, S, CO), jnp.float32),
    )(qc, qn, pcT, pn, fs, skip, uwiT, uwsT, ub, *extra_in)


def _aug(p):
    nrm = jnp.sum(p ** 2, axis=2, keepdims=True)
    z5 = jnp.zeros(p.shape[:2] + (5,), p.dtype)
    qc = jnp.concatenate([-2.0 * p, z5], axis=2)
    pcT = jnp.concatenate([p, z5], axis=2).transpose(0, 2, 1)
    return qc, nrm, pcT, nrm.transpose(0, 2, 1)


def kernel(xyz, feat, d1w1, d1b1, d1w2, d1b2, d2w1, d2b1, d2w2, d2b2,
           u1w, u1b, u2w, u2b, fw1, fb1, fw2, fb2):
    B, _, _, N = xyz.shape
    pts = xyz[:, 0].transpose(0, 2, 1)
    f0 = feat[:, 0].transpose(0, 2, 1)

    qc0, qn0, pcT0, pn0 = _aug(pts)
    pf1 = jnp.concatenate([f0, pts], axis=2)
    qsub1 = jnp.concatenate(
        [jnp.zeros((B, N, 1), jnp.float32), pts], axis=2)[:, ::4]

    nf1 = _down(qc0[:, ::4], qn0[:, ::4], pcT0, pn0, pf1, qsub1,
                d1w1.T, d1b1[None], d1w2.T, d1b2[None], nk=32, bs=256)

    xyz1 = pts[:, ::4]
    qc1, qn1, pcT1, pn1 = _aug(xyz1)
    pf2 = jnp.concatenate([nf1, xyz1], axis=2)
    qsub2 = jnp.concatenate(
        [jnp.zeros((B, 2048, 64), jnp.float32), xyz1], axis=2)[:, ::4]

    nf2 = _down(qc1[:, ::4], qn1[:, ::4], pcT1, pn1, pf2, qsub2,
                d2w1.T, d2b1[None], d2w2.T, d2b2[None], nk=32, bs=512)

    _, _, pcT2, pn2 = _aug(xyz1[:, ::4])
    f1 = _fp(qc1, qn1, pcT2, pn2, nf2, nf1,
             u1w[:, :128].T, u1w[:, 128:].T, u1b[None], bs=512)

    skip0 = jnp.concatenate([f0, jnp.zeros((B, N, 7), jnp.float32)], axis=2)
    uws2 = jnp.pad(u2w[:, 128:], ((0, 0), (0, 7)))
    fw2p = jnp.pad(fw2, ((0, 5), (0, 0)))
    fb2p = jnp.pad(fb2, (0, 5))
    out = _fp(qc0, qn0, pcT1, pn1, f1, skip0,
              u2w[:, :128].T, uws2.T, u2b[None], bs=512,
              head=(fw1.T, fb1[None], fw2p.T, fb2p[None]))

    return out[:, :, :3].transpose(0, 2, 1)


# speedup vs baseline: 1.4169x; 1.0313x over previous
import functools

import jax
import jax.numpy as jnp
from jax.experimental import pallas as pl
from jax.experimental.pallas import tpu as pltpu

_BIG = 1e30
_HI = jax.lax.Precision.HIGHEST


def _dist(qc, qn, pcT, pn):
    cross = jnp.dot(qc, pcT, preferred_element_type=jnp.float32)
    return (qn + cross) + pn


def _down_body(nk, qc_ref, qn_ref, pcT_ref, pn_ref, pfhi_ref, pflo_ref,
               hivec_ref, seg_ref, qsub_ref, w1_ref, b1_ref, w2_ref, b2_ref,
               out_ref, d_ref):
    d_ref[...] = _dist(qc_ref[0], qn_ref[0], pcT_ref[0], pn_ref[0])
    bs, n = d_ref.shape
    iota = jax.lax.broadcasted_iota(jnp.int32, d_ref.shape, 1).astype(jnp.float32)
    iota128 = jax.lax.broadcasted_iota(jnp.int32, (bs, 128), 1)
    qsub = qsub_ref[0]
    pfhi = pfhi_ref[0]
    pflo = pflo_ref[0]
    hivec = hivec_ref[...]
    seg = seg_ref[...]
    w1 = w1_ref[...]
    b1 = b1_ref[...]
    w2 = w2_ref[...]
    b2 = b2_ref[...]

    def body(_, carry):
        acc, m = carry
        dd = d_ref[...]
        ti = jnp.where(dd <= m, iota, jnp.float32(n))
        j = jnp.min(ti, axis=1, keepdims=True)
        masked = jnp.where(ti == j, _BIG, dd)
        d_ref[...] = masked
        m = jnp.min(masked, axis=1, keepdims=True)
        ji = j.astype(jnp.int32)
        jhi = jax.lax.shift_right_logical(ji, 7)
        jlo = jax.lax.bitwise_and(ji, 127)
        oh_lo = (iota128 == jlo).astype(jnp.float32)
        cm = hivec == jhi
        thi = jnp.where(cm, jnp.dot(oh_lo, pfhi,
                                    preferred_element_type=jnp.float32), 0.0)
        tlo = jnp.where(cm, jnp.dot(oh_lo, pflo,
                                    preferred_element_type=jnp.float32), 0.0)
        g = (jnp.dot(thi, seg, preferred_element_type=jnp.float32)
             + jnp.dot(tlo, seg, preferred_element_type=jnp.float32)) - qsub
        h = jnp.maximum(jnp.dot(g, w1, preferred_element_type=jnp.float32) + b1, 0.0)
        h = jnp.maximum(jnp.dot(h, w2, preferred_element_type=jnp.float32) + b2, 0.0)
        return jnp.maximum(acc, h), m

    acc0 = jnp.zeros(out_ref.shape[1:], jnp.float32)
    m0 = jnp.min(d_ref[...], axis=1, keepdims=True)
    out_ref[0] = jax.lax.fori_loop(0, nk, body, (acc0, m0))[0]


def _down(qc, qn, pcT, pn, pf, qsub, w1T, b1, w2T, b2, nk, bs):
    B, S, _ = qc.shape
    N, CF = pf.shape[1], pf.shape[2]
    C1, C2 = w1T.shape[1], w2T.shape[1]
    nc = N // 128
    pf2d = pf.reshape(B, nc, 128, CF).transpose(0, 2, 1, 3).reshape(B, 128, nc * CF)
    pfhi = pf2d.astype(jnp.bfloat16).astype(jnp.float32)
    pflo = pf2d - pfhi
    hivec = (jnp.arange(nc * CF, dtype=jnp.int32) // CF)[None]
    seg = (jnp.arange(nc * CF)[:, None] % CF ==
           jnp.arange(CF)[None, :]).astype(jnp.float32)
    return pl.pallas_call(
        functools.partial(_down_body, nk),
        grid=(B, S // bs),
        in_specs=[
            pl.BlockSpec((1, bs, 8), lambda b, s: (b, s, 0)),
            pl.BlockSpec((1, bs, 1), lambda b, s: (b, s, 0)),
            pl.BlockSpec((1, 8, N), lambda b, s: (b, 0, 0)),
            pl.BlockSpec((1, 1, N), lambda b, s: (b, 0, 0)),
            pl.BlockSpec((1, 128, nc * CF), lambda b, s: (b, 0, 0)),
            pl.BlockSpec((1, 128, nc * CF), lambda b, s: (b, 0, 0)),
            pl.BlockSpec((1, nc * CF), lambda b, s: (0, 0)),
            pl.BlockSpec((nc * CF, CF), lambda b, s: (0, 0)),
            pl.BlockSpec((1, bs, CF), lambda b, s: (b, s, 0)),
            pl.BlockSpec((CF, C1), lambda b, s: (0, 0)),
            pl.BlockSpec((1, C1), lambda b, s: (0, 0)),
            pl.BlockSpec((C1, C2), lambda b, s: (0, 0)),
            pl.BlockSpec((1, C2), lambda b, s: (0, 0)),
        ],
        out_specs=pl.BlockSpec((1, bs, C2), lambda b, s: (b, s, 0)),
        out_shape=jax.ShapeDtypeStruct((B, S, C2), jnp.float32),
        scratch_shapes=[pltpu.VMEM((bs, N), jnp.float32)],
    )(qc, qn, pcT, pn, pfhi, pflo, hivec, seg, qsub, w1T, b1, w2T, b2)


def _fp_interp(qc_ref, qn_ref, pcT_ref, pn_ref, fshi_ref, fslo_ref):
    d = _dist(qc_ref[0], qn_ref[0], pcT_ref[0], pn_ref[0])
    n = d.shape[1]
    iota = jax.lax.broadcasted_iota(jnp.int32, d.shape, 1).astype(jnp.float32)
    ws = jnp.zeros_like(d)
    wtot = jnp.zeros((d.shape[0], 1), jnp.float32)
    m = jnp.min(d, axis=1, keepdims=True)
    for t in range(3):
        ti = jnp.where(d <= m, iota, jnp.float32(n))
        j = jnp.min(ti, axis=1, keepdims=True)
        oh = (ti == j).astype(jnp.float32)
        w = 1.0 / jnp.maximum(m, 1e-10)
        ws = ws + w * oh
        wtot = wtot + w
        if t < 2:
            d = d + oh * _BIG
            m = jnp.min(d, axis=1, keepdims=True)
    wshi = ws.astype(jnp.bfloat16).astype(jnp.float32)
    wslo = ws - wshi
    interp = (jnp.dot(wshi, fshi_ref[0], preferred_element_type=jnp.float32)
              + jnp.dot(wshi, fslo_ref[0], preferred_element_type=jnp.float32)
              + jnp.dot(wslo, fshi_ref[0], preferred_element_type=jnp.float32))
    return interp / wtot


def _fp_body(qc_ref, qn_ref, pcT_ref, pn_ref, fshi_ref, fslo_ref, skip_ref,
             uwi_ref, uws_ref, ub_ref, out_ref):
    interp = _fp_interp(qc_ref, qn_ref, pcT_ref, pn_ref, fshi_ref, fslo_ref)
    f = (jnp.dot(interp, uwi_ref[...], preferred_element_type=jnp.float32)
         + jnp.dot(skip_ref[0], uws_ref[...], preferred_element_type=jnp.float32)
         + ub_ref[...])
    out_ref[0] = jnp.maximum(f, 0.0)


def _fp_head_body(qc_ref, qn_ref, pcT_ref, pn_ref, fshi_ref, fslo_ref,
                  skip_ref, uwi_ref, uws_ref, ub_ref,
                  fw1_ref, fb1_ref, fw2_ref, fb2_ref, out_ref):
    interp = _fp_interp(qc_ref, qn_ref, pcT_ref, pn_ref, fshi_ref, fslo_ref)
    f = (jnp.dot(interp, uwi_ref[...], preferred_element_type=jnp.float32)
         + jnp.dot(skip_ref[0], uws_ref[...], preferred_element_type=jnp.float32)
         + ub_ref[...])
    f = jnp.maximum(f, 0.0)
    h = jnp.maximum(jnp.dot(f, fw1_ref[...], preferred_element_type=jnp.float32)
                    + fb1_ref[...], 0.0)
    out_ref[0] = (jnp.dot(h, fw2_ref[...], preferred_element_type=jnp.float32)
                  + fb2_ref[...])


def _fp(qc, qn, pcT, pn, fs, skip, uwiT, uwsT, ub, bs, head=None):
    B, S, _ = qc.shape
    Ns, C = fs.shape[1], fs.shape[2]
    CS = skip.shape[2]
    CO = uwiT.shape[1]
    extra_in, extra_specs = (), []
    body = _fp_body
    if head is not None:
        fw1T, fb1, fw2T, fb2 = head
        CO = fw2T.shape[1]
        extra_in = (fw1T, fb1, fw2T, fb2)
        extra_specs = [
            pl.BlockSpec(fw1T.shape, lambda b, s: (0, 0)),
            pl.BlockSpec(fb1.shape, lambda b, s: (0, 0)),
            pl.BlockSpec(fw2T.shape, lambda b, s: (0, 0)),
            pl.BlockSpec(fb2.shape, lambda b, s: (0, 0)),
        ]
        body = _fp_head_body
    fshi = fs.astype(jnp.bfloat16).astype(jnp.float32)
    fslo = fs - fshi
    return pl.pallas_call(
        body,
        grid=(B, S // bs),
        in_specs=[
            pl.BlockSpec((1, bs, 8), lambda b, s: (b, s, 0)),
            pl.BlockSpec((1, bs, 1), lambda b, s: (b, s, 0)),
            pl.BlockSpec((1, 8, Ns), lambda b, s: (b, 0, 0)),
            pl.BlockSpec((1, 1, Ns), lambda b, s: (b, 0, 0)),
            pl.BlockSpec((1, Ns, C), lambda b, s: (b, 0, 0)),
            pl.BlockSpec((1, Ns, C), lambda b, s: (b, 0, 0)),
            pl.BlockSpec((1, bs, CS), lambda b, s: (b, s, 0)),
            pl.BlockSpec(uwiT.shape, lambda b, s: (0, 0)),
            pl.BlockSpec(uwsT.shape, lambda b, s: (0, 0)),
            pl.BlockSpec(ub.shape, lambda b, s: (0, 0)),
        ] + extra_specs,
        out_specs=pl.BlockSpec((1, bs, CO), lambda b, s: (b, s, 0)),
        out_shape=jax.ShapeDtypeStruct((B, S, CO), jnp.float32),
    )(qc, qn, pcT, pn, fshi, fslo, skip, uwiT, uwsT, ub, *extra_in)


def _aug(p):
    nrm = jnp.sum(p ** 2, axis=2, keepdims=True)
    z5 = jnp.zeros(p.shape[:2] + (5,), p.dtype)
    qc = jnp.concatenate([-2.0 * p, z5], axis=2)
    pcT = jnp.concatenate([p, z5], axis=2).transpose(0, 2, 1)
    return qc, nrm, pcT, nrm.transpose(0, 2, 1)


def kernel(xyz, feat, d1w1, d1b1, d1w2, d1b2, d2w1, d2b1, d2w2, d2b2,
           u1w, u1b, u2w, u2b, fw1, fb1, fw2, fb2):
    B, _, _, N = xyz.shape
    pts = xyz[:, 0].transpose(0, 2, 1)
    f0 = feat[:, 0].transpose(0, 2, 1)

    qc0, qn0, pcT0, pn0 = _aug(pts)
    pf1 = jnp.concatenate([f0, pts], axis=2)
    qsub1 = jnp.concatenate(
        [jnp.zeros((B, N, 1), jnp.float32), pts], axis=2)[:, ::4]

    nf1 = _down(qc0[:, ::4], qn0[:, ::4], pcT0, pn0, pf1, qsub1,
                d1w1.T, d1b1[None], d1w2.T, d1b2[None], nk=32, bs=256)

    xyz1 = pts[:, ::4]
    qc1, qn1, pcT1, pn1 = _aug(xyz1)
    pf2 = jnp.concatenate([nf1, xyz1], axis=2)
    qsub2 = jnp.concatenate(
        [jnp.zeros((B, 2048, 64), jnp.float32), xyz1], axis=2)[:, ::4]

    nf2 = _down(qc1[:, ::4], qn1[:, ::4], pcT1, pn1, pf2, qsub2,
                d2w1.T, d2b1[None], d2w2.T, d2b2[None], nk=32, bs=512)

    _, _, pcT2, pn2 = _aug(xyz1[:, ::4])
    f1 = _fp(qc1, qn1, pcT2, pn2, nf2, nf1,
             u1w[:, :128].T, u1w[:, 128:].T, u1b[None], bs=512)

    skip0 = jnp.concatenate([f0, jnp.zeros((B, N, 7), jnp.float32)], axis=2)
    uws2 = jnp.pad(u2w[:, 128:], ((0, 0), (0, 7)))
    fw2p = jnp.pad(fw2, ((0, 5), (0, 0)))
    fb2p = jnp.pad(fb2, (0, 5))
    out = _fp(qc0, qn0, pcT1, pn1, f1, skip0,
              u2w[:, :128].T, uws2.T, u2b[None], bs=512,
              head=(fw1.T, fb1[None], fw2p.T, fb2p[None]))

    return out[:, :, :3].transpose(0, 2, 1)


# speedup vs baseline: 1.4593x; 1.0300x over previous
import functools

import jax
import jax.numpy as jnp
from jax.experimental import pallas as pl
from jax.experimental.pallas import tpu as pltpu

_BIG = 1e30
_HI = jax.lax.Precision.HIGHEST


def _dist(qc, qn, pcT, pn):
    cross = jnp.dot(qc, pcT, preferred_element_type=jnp.float32)
    return (qn + cross) + pn


def _down_body(nk, qc_ref, qn_ref, pcT_ref, pn_ref, pfhi_ref, pflo_ref,
               hivec_ref, qsub_ref, w1_ref, b1_ref, w2_ref, b2_ref,
               out_ref, d_ref):
    d_ref[...] = _dist(qc_ref[0], qn_ref[0], pcT_ref[0], pn_ref[0])
    bs, n = d_ref.shape
    iota = jax.lax.broadcasted_iota(jnp.int32, d_ref.shape, 1).astype(jnp.float32)
    iota128 = jax.lax.broadcasted_iota(jnp.int32, (bs, 128), 1)
    qsub = qsub_ref[0]
    pfhi = pfhi_ref[0]
    pflo = pflo_ref[0]
    hivec = hivec_ref[...]
    w1 = w1_ref[...]
    b1 = b1_ref[...]
    w2 = w2_ref[...]
    b2 = b2_ref[...]

    def body(_, carry):
        acc, m = carry
        dd = d_ref[...]
        ti = jnp.where(dd <= m, iota, jnp.float32(n))
        j = jnp.min(ti, axis=1, keepdims=True)
        masked = jnp.where(ti == j, _BIG, dd)
        d_ref[...] = masked
        m = jnp.min(masked, axis=1, keepdims=True)
        ji = j.astype(jnp.int32)
        jhi = jax.lax.shift_right_logical(ji, 7)
        jlo = jax.lax.bitwise_and(ji, 127)
        oh_lo = (iota128 == jlo).astype(jnp.float32)
        cm = hivec == jhi
        thi = jnp.where(cm, jnp.dot(oh_lo, pfhi,
                                    preferred_element_type=jnp.float32), 0.0)
        tlo = jnp.where(cm, jnp.dot(oh_lo, pflo,
                                    preferred_element_type=jnp.float32), 0.0)
        t = thi + tlo
        w = t.shape[1]
        cf = qsub.shape[1]
        while w > cf:
            w //= 2
            t = t[:, :w] + t[:, w:2 * w]
        g = t - qsub
        h = jnp.maximum(jnp.dot(g, w1, preferred_element_type=jnp.float32) + b1, 0.0)
        h = jnp.maximum(jnp.dot(h, w2, preferred_element_type=jnp.float32) + b2, 0.0)
        return jnp.maximum(acc, h), m

    acc0 = jnp.zeros(out_ref.shape[1:], jnp.float32)
    m0 = jnp.min(d_ref[...], axis=1, keepdims=True)
    out_ref[0] = jax.lax.fori_loop(0, nk, body, (acc0, m0))[0]


def _down(qc, qn, pcT, pn, pf, qsub, w1T, b1, w2T, b2, nk, bs):
    B, S, _ = qc.shape
    N, CF = pf.shape[1], pf.shape[2]
    C1, C2 = w1T.shape[1], w2T.shape[1]
    nc = N // 128
    pf2d = pf.reshape(B, nc, 128, CF).transpose(0, 2, 1, 3).reshape(B, 128, nc * CF)
    pfhi = pf2d.astype(jnp.bfloat16).astype(jnp.float32)
    pflo = pf2d - pfhi
    hivec = (jnp.arange(nc * CF, dtype=jnp.int32) // CF)[None]
    return pl.pallas_call(
        functools.partial(_down_body, nk),
        grid=(B, S // bs),
        in_specs=[
            pl.BlockSpec((1, bs, 8), lambda b, s: (b, s, 0)),
            pl.BlockSpec((1, bs, 1), lambda b, s: (b, s, 0)),
            pl.BlockSpec((1, 8, N), lambda b, s: (b, 0, 0)),
            pl.BlockSpec((1, 1, N), lambda b, s: (b, 0, 0)),
            pl.BlockSpec((1, 128, nc * CF), lambda b, s: (b, 0, 0)),
            pl.BlockSpec((1, 128, nc * CF), lambda b, s: (b, 0, 0)),
            pl.BlockSpec((1, nc * CF), lambda b, s: (0, 0)),
            pl.BlockSpec((1, bs, CF), lambda b, s: (b, s, 0)),
            pl.BlockSpec((CF, C1), lambda b, s: (0, 0)),
            pl.BlockSpec((1, C1), lambda b, s: (0, 0)),
            pl.BlockSpec((C1, C2), lambda b, s: (0, 0)),
            pl.BlockSpec((1, C2), lambda b, s: (0, 0)),
        ],
        out_specs=pl.BlockSpec((1, bs, C2), lambda b, s: (b, s, 0)),
        out_shape=jax.ShapeDtypeStruct((B, S, C2), jnp.float32),
        scratch_shapes=[pltpu.VMEM((bs, N), jnp.float32)],
    )(qc, qn, pcT, pn, pfhi, pflo, hivec, qsub, w1T, b1, w2T, b2)


def _fp_interp(qc_ref, qn_ref, pcT_ref, pn_ref, fshi_ref, fslo_ref):
    d = _dist(qc_ref[0], qn_ref[0], pcT_ref[0], pn_ref[0])
    n = d.shape[1]
    iota = jax.lax.broadcasted_iota(jnp.int32, d.shape, 1).astype(jnp.float32)
    ws = jnp.zeros_like(d)
    wtot = jnp.zeros((d.shape[0], 1), jnp.float32)
    m = jnp.min(d, axis=1, keepdims=True)
    for t in range(3):
        ti = jnp.where(d <= m, iota, jnp.float32(n))
        j = jnp.min(ti, axis=1, keepdims=True)
        oh = (ti == j).astype(jnp.float32)
        w = 1.0 / jnp.maximum(m, 1e-10)
        ws = ws + w * oh
        wtot = wtot + w
        if t < 2:
            d = d + oh * _BIG
            m = jnp.min(d, axis=1, keepdims=True)
    wshi = ws.astype(jnp.bfloat16).astype(jnp.float32)
    wslo = ws - wshi
    interp = (jnp.dot(wshi, fshi_ref[0], preferred_element_type=jnp.float32)
              + jnp.dot(wshi, fslo_ref[0], preferred_element_type=jnp.float32)
              + jnp.dot(wslo, fshi_ref[0], preferred_element_type=jnp.float32)
              + jnp.dot(wslo, fslo_ref[0], preferred_element_type=jnp.float32))
    return interp / wtot


def _fp_body(qc_ref, qn_ref, pcT_ref, pn_ref, fshi_ref, fslo_ref, skip_ref,
             uwi_ref, uws_ref, ub_ref, out_ref):
    interp = _fp_interp(qc_ref, qn_ref, pcT_ref, pn_ref, fshi_ref, fslo_ref)
    f = (jnp.dot(interp, uwi_ref[...], preferred_element_type=jnp.float32)
         + jnp.dot(skip_ref[0], uws_ref[...], preferred_element_type=jnp.float32)
         + ub_ref[...])
    out_ref[0] = jnp.maximum(f, 0.0)


def _fp_head_body(qc_ref, qn_ref, pcT_ref, pn_ref, fshi_ref, fslo_ref,
                  skip_ref, uwi_ref, uws_ref, ub_ref,
                  fw1_ref, fb1_ref, fw2_ref, fb2_ref, out_ref):
    interp = _fp_interp(qc_ref, qn_ref, pcT_ref, pn_ref, fshi_ref, fslo_ref)
    f = (jnp.dot(interp, uwi_ref[...], preferred_element_type=jnp.float32)
         + jnp.dot(skip_ref[0], uws_ref[...], preferred_element_type=jnp.float32)
         + ub_ref[...])
    f = jnp.maximum(f, 0.0)
    h = jnp.maximum(jnp.dot(f, fw1_ref[...], preferred_element_type=jnp.float32)
                    + fb1_ref[...], 0.0)
    out_ref[0] = (jnp.dot(h, fw2_ref[...], preferred_element_type=jnp.float32)
                  + fb2_ref[...])


def _fp(qc, qn, pcT, pn, fs, skip, uwiT, uwsT, ub, bs, head=None):
    B, S, _ = qc.shape
    Ns, C = fs.shape[1], fs.shape[2]
    CS = skip.shape[2]
    CO = uwiT.shape[1]
    extra_in, extra_specs = (), []
    body = _fp_body
    if head is not None:
        fw1T, fb1, fw2T, fb2 = head
        CO = fw2T.shape[1]
        extra_in = (fw1T, fb1, fw2T, fb2)
        extra_specs = [
            pl.BlockSpec(fw1T.shape, lambda b, s: (0, 0)),
            pl.BlockSpec(fb1.shape, lambda b, s: (0, 0)),
            pl.BlockSpec(fw2T.shape, lambda b, s: (0, 0)),
            pl.BlockSpec(fb2.shape, lambda b, s: (0, 0)),
        ]
        body = _fp_head_body
    fshi = fs.astype(jnp.bfloat16).astype(jnp.float32)
    fslo = fs - fshi
    return pl.pallas_call(
        body,
        grid=(B, S // bs),
        in_specs=[
            pl.BlockSpec((1, bs, 8), lambda b, s: (b, s, 0)),
            pl.BlockSpec((1, bs, 1), lambda b, s: (b, s, 0)),
            pl.BlockSpec((1, 8, Ns), lambda b, s: (b, 0, 0)),
            pl.BlockSpec((1, 1, Ns), lambda b, s: (b, 0, 0)),
            pl.BlockSpec((1, Ns, C), lambda b, s: (b, 0, 0)),
            pl.BlockSpec((1, Ns, C), lambda b, s: (b, 0, 0)),
            pl.BlockSpec((1, bs, CS), lambda b, s: (b, s, 0)),
            pl.BlockSpec(uwiT.shape, lambda b, s: (0, 0)),
            pl.BlockSpec(uwsT.shape, lambda b, s: (0, 0)),
            pl.BlockSpec(ub.shape, lambda b, s: (0, 0)),
        ] + extra_specs,
        out_specs=pl.BlockSpec((1, bs, CO), lambda b, s: (b, s, 0)),
        out_shape=jax.ShapeDtypeStruct((B, S, CO), jnp.float32),
    )(qc, qn, pcT, pn, fshi, fslo, skip, uwiT, uwsT, ub, *extra_in)


def _aug(p):
    nrm = jnp.sum(p ** 2, axis=2, keepdims=True)
    z5 = jnp.zeros(p.shape[:2] + (5,), p.dtype)
    qc = jnp.concatenate([-2.0 * p, z5], axis=2)
    pcT = jnp.concatenate([p, z5], axis=2).transpose(0, 2, 1)
    return qc, nrm, pcT, nrm.transpose(0, 2, 1)


def kernel(xyz, feat, d1w1, d1b1, d1w2, d1b2, d2w1, d2b1, d2w2, d2b2,
           u1w, u1b, u2w, u2b, fw1, fb1, fw2, fb2):
    B, _, _, N = xyz.shape
    pts = xyz[:, 0].transpose(0, 2, 1)
    f0 = feat[:, 0].transpose(0, 2, 1)

    qc0, qn0, pcT0, pn0 = _aug(pts)
    pf1 = jnp.concatenate([f0, pts], axis=2)
    qsub1 = jnp.concatenate(
        [jnp.zeros((B, N, 1), jnp.float32), pts], axis=2)[:, ::4]

    nf1 = _down(qc0[:, ::4], qn0[:, ::4], pcT0, pn0, pf1, qsub1,
                d1w1.T, d1b1[None], d1w2.T, d1b2[None], nk=32, bs=256)

    xyz1 = pts[:, ::4]
    qc1, qn1, pcT1, pn1 = _aug(xyz1)
    pf2 = jnp.concatenate([nf1, xyz1], axis=2)
    qsub2 = jnp.concatenate(
        [jnp.zeros((B, 2048, 64), jnp.float32), xyz1], axis=2)[:, ::4]

    nf2 = _down(qc1[:, ::4], qn1[:, ::4], pcT1, pn1, pf2, qsub2,
                d2w1.T, d2b1[None], d2w2.T, d2b2[None], nk=32, bs=512)

    _, _, pcT2, pn2 = _aug(xyz1[:, ::4])
    f1 = _fp(qc1, qn1, pcT2, pn2, nf2, nf1,
             u1w[:, :128].T, u1w[:, 128:].T, u1b[None], bs=512)

    skip0 = jnp.concatenate([f0, jnp.zeros((B, N, 7), jnp.float32)], axis=2)
    uws2 = jnp.pad(u2w[:, 128:], ((0, 0), (0, 7)))
    fw2p = jnp.pad(fw2, ((0, 5), (0, 0)))
    fb2p = jnp.pad(fb2, (0, 5))
    out = _fp(qc0, qn0, pcT1, pn1, f1, skip0,
              u2w[:, :128].T, uws2.T, u2b[None], bs=512,
              head=(fw1.T, fb1[None], fw2p.T, fb2p[None]))

    return out[:, :, :3].transpose(0, 2, 1)
